# Initial kernel scaffold; baseline (speedup 1.0000x reference)
import jax
import jax.numpy as jnp
from jax.experimental import pallas as pl


def kernel(edge_index, x, W1, b1, W2, b2):
    raise NotImplementedError("write your pallas kernel here")


# speedup vs baseline: 14.1754x; 14.1754x over previous
import functools

import jax
import jax.numpy as jnp
from jax import lax
from jax.experimental import pallas as pl
from jax.experimental.pallas import tpu as pltpu
from jax.experimental.pallas import tpu_sc as plsc

NC = 2
NS = 16
NW = NC * NS

NHID = 128
HHALF = NHID // 2

CHUNK = 128
NCHUNK = 80
EDGES_PER_TILE = NCHUNK * CHUNK
E_PAD = NW * EDGES_PER_TILE

NPAD = 10112
ROWS_PER_TILE = NPAD // NS
DW = 16

RB = 2000

_MESH = plsc.VectorSubcoreMesh(
    core_axis_name="c", subcore_axis_name="s", num_cores=NC, num_subcores=NS
)


@functools.partial(
    pl.kernel,
    out_type=jax.ShapeDtypeStruct((NC, NPAD, DW), jnp.float32),
    mesh=_MESH,
    scratch_types=[
        pltpu.VMEM((NCHUNK, CHUNK), jnp.int32),
        pltpu.VMEM((CHUNK, DW), jnp.float32),
        pltpu.VMEM((ROWS_PER_TILE, DW), jnp.float32),
        pltpu.VMEM_SHARED((NPAD, DW), jnp.float32),
    ],
    compiler_params=pltpu.CompilerParams(use_tc_tiling_on_sc=False),
)
def _deg_kernel(dst_hbm, out_hbm, dst_v, ones_v, stage_v, deg_sh):
    c = lax.axis_index("c")
    s = lax.axis_index("s")
    wid = c * NS + s
    base = s * ROWS_PER_TILE

    @pl.loop(0, CHUNK)
    def _(i):
        ones_v[i, :] = jnp.ones((DW,), jnp.float32)

    @pl.loop(0, ROWS_PER_TILE)
    def _(i):
        stage_v[i, :] = jnp.zeros((DW,), jnp.float32)

    pltpu.sync_copy(stage_v, deg_sh.at[pl.ds(base, ROWS_PER_TILE)])
    plsc.subcore_barrier()

    pltpu.sync_copy(dst_hbm.at[wid], dst_v)

    @pl.loop(0, NCHUNK)
    def _(g):
        pltpu.sync_copy(ones_v, deg_sh.at[dst_v.at[g]], add=True)

    plsc.subcore_barrier()
    pltpu.sync_copy(deg_sh.at[pl.ds(base, ROWS_PER_TILE)], stage_v)
    pltpu.sync_copy(stage_v, out_hbm.at[c, pl.ds(base, ROWS_PER_TILE)])


@functools.partial(
    pl.kernel,
    out_type=jax.ShapeDtypeStruct((2, NC, NPAD, HHALF), jnp.float32),
    mesh=_MESH,
    scratch_types=[
        pltpu.VMEM((NCHUNK, CHUNK), jnp.int32),
        pltpu.VMEM((NCHUNK, CHUNK), jnp.int32),
        pltpu.VMEM((2, CHUNK, HHALF), jnp.float32),
        pltpu.SemaphoreType.DMA,
        pltpu.SemaphoreType.DMA,
        pltpu.VMEM_SHARED((NPAD, HHALF), jnp.float32),
    ],
    compiler_params=pltpu.CompilerParams(use_tc_tiling_on_sc=False),
)
def _agg_kernel(hs0_hbm, hs1_hbm, src_hbm, dst_hbm, out_hbm,
                src_v, dst_v, buf_v, sem0, sem1, agg_sh):
    c = lax.axis_index("c")
    s = lax.axis_index("s")
    wid = c * NS + s
    base = s * ROWS_PER_TILE
    sems = (sem0, sem1)
    rem = ROWS_PER_TILE - 4 * CHUNK

    pltpu.sync_copy(src_hbm.at[wid], src_v)
    pltpu.sync_copy(dst_hbm.at[wid], dst_v)

    for half, hs_hbm in ((0, hs0_hbm), (1, hs1_hbm)):
        @pl.loop(0, CHUNK)
        def _(i):
            for j in range(HHALF // 16):
                buf_v[0, i, pl.ds(j * 16, 16)] = jnp.zeros((16,), jnp.float32)

        for k in range(4):
            pltpu.sync_copy(buf_v.at[0],
                            agg_sh.at[pl.ds(base + k * CHUNK, CHUNK)])
        pltpu.sync_copy(buf_v.at[0, pl.ds(0, rem)],
                        agg_sh.at[pl.ds(base + 4 * CHUNK, rem)])
        plsc.subcore_barrier()

        pltpu.async_copy(hs_hbm.at[src_v.at[0]], buf_v.at[0], sem0)
        pltpu.async_copy(hs_hbm.at[src_v.at[1]], buf_v.at[1], sem1)

        @pl.loop(0, NCHUNK - 2, step=2)
        def _(g):
            for b in range(2):
                cur = g + b
                pltpu.make_async_copy(hs_hbm.at[src_v.at[cur]], buf_v.at[b],
                                      sems[b]).wait()
                pltpu.sync_copy(buf_v.at[b], agg_sh.at[dst_v.at[cur]],
                                add=True)
                pltpu.async_copy(hs_hbm.at[src_v.at[cur + 2]], buf_v.at[b],
                                 sems[b])

        for b in range(2):
            cur = NCHUNK - 2 + b
            pltpu.make_async_copy(hs_hbm.at[src_v.at[cur]], buf_v.at[b],
                                  sems[b]).wait()
            pltpu.sync_copy(buf_v.at[b], agg_sh.at[dst_v.at[cur]], add=True)

        plsc.subcore_barrier()
        pltpu.sync_copy(agg_sh.at[pl.ds(base, ROWS_PER_TILE)],
                        out_hbm.at[half, c, pl.ds(base, ROWS_PER_TILE)])
        plsc.subcore_barrier()


def _hs_body(x_ref, w_ref, degp_ref, hs0_ref, hs1_ref):
    h = jnp.dot(x_ref[...], w_ref[...], preferred_element_type=jnp.float32)
    dp = degp_ref[...]
    deg = dp[0] + dp[1] + 1.0
    dinv = lax.rsqrt(deg)[:, 0:1]
    hs = h * dinv
    hs0_ref[...] = hs[:, :HHALF]
    hs1_ref[...] = hs[:, HHALF:]


def _out_body(aggp_ref, hs0_ref, hs1_ref, degp_ref, w2_ref, b1_ref, b2_ref,
              o_ref):
    dp = degp_ref[...]
    deg = dp[0] + dp[1] + 1.0
    dinv = lax.rsqrt(deg)[:, 0:1]
    ap = aggp_ref[...]
    agg = jnp.concatenate([ap[0, 0] + ap[0, 1], ap[1, 0] + ap[1, 1]], axis=1)
    hs = jnp.concatenate([hs0_ref[...], hs1_ref[...]], axis=1)
    t = (agg + hs) * dinv + b1_ref[...]
    o_ref[...] = jnp.dot(t, w2_ref[...], preferred_element_type=jnp.float32) \
        + b2_ref[...]


def kernel(edge_index, x, W1, b1, W2, b2):
    n, f = x.shape
    e = edge_index.shape[1]
    nclass = W2.shape[1]

    ei = edge_index.astype(jnp.int32)
    src = jnp.pad(ei[0], (0, E_PAD - e)).reshape(NW, NCHUNK, CHUNK)
    dst = jnp.pad(ei[1], (0, E_PAD - e),
                  constant_values=n).reshape(NW, NCHUNK, CHUNK)

    degp = _deg_kernel(dst)

    hs0, hs1 = pl.pallas_call(
        _hs_body,
        grid=(n // RB,),
        in_specs=[
            pl.BlockSpec((RB, f), lambda i: (i, 0)),
            pl.BlockSpec((f, NHID), lambda i: (0, 0)),
            pl.BlockSpec((NC, RB, DW), lambda i: (0, i, 0)),
        ],
        out_specs=[
            pl.BlockSpec((RB, HHALF), lambda i: (i, 0)),
            pl.BlockSpec((RB, HHALF), lambda i: (i, 0)),
        ],
        out_shape=[
            jax.ShapeDtypeStruct((n, HHALF), jnp.float32),
            jax.ShapeDtypeStruct((n, HHALF), jnp.float32),
        ],
    )(x, W1, degp)

    aggp = _agg_kernel(hs0, hs1, src, dst)

    w2p = jnp.zeros((NHID, 128), jnp.float32).at[:, :nclass].set(W2)
    b2p = jnp.zeros((128,), jnp.float32).at[:nclass].set(b2)

    out = pl.pallas_call(
        _out_body,
        grid=(n // RB,),
        in_specs=[
            pl.BlockSpec((2, NC, RB, HHALF), lambda i: (0, 0, i, 0)),
            pl.BlockSpec((RB, HHALF), lambda i: (i, 0)),
            pl.BlockSpec((RB, HHALF), lambda i: (i, 0)),
            pl.BlockSpec((NC, RB, DW), lambda i: (0, i, 0)),
            pl.BlockSpec((NHID, 128), lambda i: (0, 0)),
            pl.BlockSpec((NHID,), lambda i: (0,)),
            pl.BlockSpec((128,), lambda i: (0,)),
        ],
        out_specs=pl.BlockSpec((RB, 128), lambda i: (i, 0)),
        out_shape=jax.ShapeDtypeStruct((n, 128), jnp.float32),
    )(aggp, hs0, hs1, degp, w2p, b1, b2p)

    return out[:, :nclass]


# speedup vs baseline: 24.5572x; 1.7324x over previous
import functools

import jax
import jax.numpy as jnp
from jax import lax
from jax.experimental import pallas as pl
from jax.experimental.pallas import tpu as pltpu
from jax.experimental.pallas import tpu_sc as plsc

NC = 2
NS = 16
NW = NC * NS

NHID = 128
NQ = 4
HQ = NHID // NQ

CHUNK = 128
NCHUNK = 80
EDGES_PER_TILE = NCHUNK * CHUNK
E_PAD = NW * EDGES_PER_TILE

NPAD = 10112
ROWS_PER_TILE = NPAD // NS
N_NODES = 10000
HS_ROWS_PER_TILE = N_NODES // NS
DW = 16

RB = 2000

_MESH = plsc.VectorSubcoreMesh(
    core_axis_name="c", subcore_axis_name="s", num_cores=NC, num_subcores=NS
)


@functools.partial(
    pl.kernel,
    out_type=jax.ShapeDtypeStruct((NC, NPAD, DW), jnp.float32),
    mesh=_MESH,
    scratch_types=[
        pltpu.VMEM((NCHUNK, CHUNK), jnp.int32),
        pltpu.VMEM((CHUNK, DW), jnp.float32),
        pltpu.VMEM((ROWS_PER_TILE, DW), jnp.float32),
        pltpu.VMEM_SHARED((NPAD, DW), jnp.float32),
    ],
    compiler_params=pltpu.CompilerParams(use_tc_tiling_on_sc=False),
)
def _deg_kernel(dst_hbm, out_hbm, dst_v, ones_v, stage_v, deg_sh):
    c = lax.axis_index("c")
    s = lax.axis_index("s")
    wid = c * NS + s
    base = s * ROWS_PER_TILE

    @pl.loop(0, CHUNK)
    def _(i):
        ones_v[i, :] = jnp.ones((DW,), jnp.float32)

    @pl.loop(0, ROWS_PER_TILE)
    def _(i):
        stage_v[i, :] = jnp.zeros((DW,), jnp.float32)

    pltpu.sync_copy(stage_v, deg_sh.at[pl.ds(base, ROWS_PER_TILE)])
    plsc.subcore_barrier()

    pltpu.sync_copy(dst_hbm.at[wid], dst_v)

    @pl.loop(0, NCHUNK)
    def _(g):
        pltpu.sync_copy(ones_v, deg_sh.at[dst_v.at[g]], add=True)

    plsc.subcore_barrier()
    pltpu.sync_copy(deg_sh.at[pl.ds(base, ROWS_PER_TILE)], stage_v)
    pltpu.sync_copy(stage_v, out_hbm.at[c, pl.ds(base, ROWS_PER_TILE)])


@functools.partial(
    pl.kernel,
    out_type=jax.ShapeDtypeStruct((NQ, NC, NPAD, HQ), jnp.float32),
    mesh=_MESH,
    scratch_types=[
        pltpu.VMEM((NCHUNK, CHUNK), jnp.int32),
        pltpu.VMEM((NCHUNK, CHUNK), jnp.int32),
        pltpu.VMEM((2, CHUNK, HQ), jnp.float32),
        pltpu.SemaphoreType.DMA,
        pltpu.SemaphoreType.DMA,
        pltpu.VMEM_SHARED((N_NODES, HQ), jnp.float32),
        pltpu.VMEM_SHARED((NPAD, HQ), jnp.float32),
    ],
    compiler_params=pltpu.CompilerParams(use_tc_tiling_on_sc=False),
)
def _agg_kernel(hs0_hbm, hs1_hbm, hs2_hbm, hs3_hbm, src_hbm, dst_hbm, out_hbm,
                src_v, dst_v, buf_v, sem0, sem1, hs_sh, agg_sh):
    c = lax.axis_index("c")
    s = lax.axis_index("s")
    wid = c * NS + s
    base = s * ROWS_PER_TILE
    hs_base = s * HS_ROWS_PER_TILE
    sems = (sem0, sem1)
    rem = ROWS_PER_TILE - 4 * CHUNK

    pltpu.sync_copy(src_hbm.at[wid], src_v)
    pltpu.sync_copy(dst_hbm.at[wid], dst_v)

    for q, hs_hbm in enumerate((hs0_hbm, hs1_hbm, hs2_hbm, hs3_hbm)):
        @pl.loop(0, CHUNK)
        def _(i):
            for j in range(HQ // 16):
                buf_v[0, i, pl.ds(j * 16, 16)] = jnp.zeros((16,), jnp.float32)

        for k in range(4):
            pltpu.sync_copy(buf_v.at[0],
                            agg_sh.at[pl.ds(base + k * CHUNK, CHUNK)])
        pltpu.sync_copy(buf_v.at[0, pl.ds(0, rem)],
                        agg_sh.at[pl.ds(base + 4 * CHUNK, rem)])

        pltpu.sync_copy(hs_hbm.at[pl.ds(hs_base, HS_ROWS_PER_TILE)],
                        hs_sh.at[pl.ds(hs_base, HS_ROWS_PER_TILE)])
        plsc.subcore_barrier()

        pltpu.async_copy(hs_sh.at[src_v.at[0]], buf_v.at[0], sem0)
        pltpu.async_copy(hs_sh.at[src_v.at[1]], buf_v.at[1], sem1)

        @pl.loop(0, NCHUNK - 2, step=2)
        def _(g):
            for b in range(2):
                cur = g + b
                pltpu.make_async_copy(hs_sh.at[src_v.at[cur]], buf_v.at[b],
                                      sems[b]).wait()
                pltpu.sync_copy(buf_v.at[b], agg_sh.at[dst_v.at[cur]],
                                add=True)
                pltpu.async_copy(hs_sh.at[src_v.at[cur + 2]], buf_v.at[b],
                                 sems[b])

        for b in range(2):
            cur = NCHUNK - 2 + b
            pltpu.make_async_copy(hs_sh.at[src_v.at[cur]], buf_v.at[b],
                                  sems[b]).wait()
            pltpu.sync_copy(buf_v.at[b], agg_sh.at[dst_v.at[cur]], add=True)

        plsc.subcore_barrier()
        pltpu.sync_copy(agg_sh.at[pl.ds(base, ROWS_PER_TILE)],
                        out_hbm.at[q, c, pl.ds(base, ROWS_PER_TILE)])
        plsc.subcore_barrier()


def _hs_body(x_ref, w_ref, degp_ref, hs0_ref, hs1_ref, hs2_ref, hs3_ref):
    h = jnp.dot(x_ref[...], w_ref[...], preferred_element_type=jnp.float32)
    dp = degp_ref[...]
    deg = dp[0] + dp[1] + 1.0
    dinv = lax.rsqrt(deg)[:, 0:1]
    hs = h * dinv
    hs0_ref[...] = hs[:, 0 * HQ:1 * HQ]
    hs1_ref[...] = hs[:, 1 * HQ:2 * HQ]
    hs2_ref[...] = hs[:, 2 * HQ:3 * HQ]
    hs3_ref[...] = hs[:, 3 * HQ:4 * HQ]


def _out_body(aggp_ref, hs0_ref, hs1_ref, hs2_ref, hs3_ref, degp_ref, w2_ref,
              b1_ref, b2_ref, o_ref):
    dp = degp_ref[...]
    deg = dp[0] + dp[1] + 1.0
    dinv = lax.rsqrt(deg)[:, 0:1]
    ap = aggp_ref[...]
    agg = jnp.concatenate([ap[q, 0] + ap[q, 1] for q in range(NQ)], axis=1)
    hs = jnp.concatenate(
        [hs0_ref[...], hs1_ref[...], hs2_ref[...], hs3_ref[...]], axis=1)
    t = (agg + hs) * dinv + b1_ref[...]
    o_ref[...] = jnp.dot(t, w2_ref[...], preferred_element_type=jnp.float32) \
        + b2_ref[...]


def kernel(edge_index, x, W1, b1, W2, b2):
    n, f = x.shape
    e = edge_index.shape[1]
    nclass = W2.shape[1]

    ei = edge_index.astype(jnp.int32)
    src = jnp.pad(ei[0], (0, E_PAD - e)).reshape(NW, NCHUNK, CHUNK)
    dst = jnp.pad(ei[1], (0, E_PAD - e),
                  constant_values=n).reshape(NW, NCHUNK, CHUNK)

    degp = _deg_kernel(dst)

    hsq = pl.pallas_call(
        _hs_body,
        grid=(n // RB,),
        in_specs=[
            pl.BlockSpec((RB, f), lambda i: (i, 0)),
            pl.BlockSpec((f, NHID), lambda i: (0, 0)),
            pl.BlockSpec((NC, RB, DW), lambda i: (0, i, 0)),
        ],
        out_specs=[pl.BlockSpec((RB, HQ), lambda i: (i, 0))] * NQ,
        out_shape=[jax.ShapeDtypeStruct((n, HQ), jnp.float32)] * NQ,
    )(x, W1, degp)

    aggp = _agg_kernel(*hsq, src, dst)

    w2p = jnp.zeros((NHID, 128), jnp.float32).at[:, :nclass].set(W2)
    b2p = jnp.zeros((128,), jnp.float32).at[:nclass].set(b2)

    out = pl.pallas_call(
        _out_body,
        grid=(n // RB,),
        in_specs=[
            pl.BlockSpec((NQ, NC, RB, HQ), lambda i: (0, 0, i, 0)),
            pl.BlockSpec((RB, HQ), lambda i: (i, 0)),
            pl.BlockSpec((RB, HQ), lambda i: (i, 0)),
            pl.BlockSpec((RB, HQ), lambda i: (i, 0)),
            pl.BlockSpec((RB, HQ), lambda i: (i, 0)),
            pl.BlockSpec((NC, RB, DW), lambda i: (0, i, 0)),
            pl.BlockSpec((NHID, 128), lambda i: (0, 0)),
            pl.BlockSpec((NHID,), lambda i: (0,)),
            pl.BlockSpec((128,), lambda i: (0,)),
        ],
        out_specs=pl.BlockSpec((RB, 128), lambda i: (i, 0)),
        out_shape=jax.ShapeDtypeStruct((n, 128), jnp.float32),
    )(aggp, *hsq, degp, w2p, b1, b2p)

    return out[:, :nclass]


# speedup vs baseline: 28.6093x; 1.1650x over previous
import functools

import jax
import jax.numpy as jnp
from jax import lax
from jax.experimental import pallas as pl
from jax.experimental.pallas import tpu as pltpu
from jax.experimental.pallas import tpu_sc as plsc

NC = 2
NS = 16
NW = NC * NS

NHID = 128
NQ = 2
HQ = NHID // NQ

CHUNK = 128
NCHUNK = 80
EDGES_PER_TILE = NCHUNK * CHUNK
E_PAD = NW * EDGES_PER_TILE

NPAD = 10112
ROWS_PER_TILE = NPAD // NS
N_NODES = 10000
HS_ROWS_PER_TILE = N_NODES // NS
DW = 16

RB = 2000

_MESH = plsc.VectorSubcoreMesh(
    core_axis_name="c", subcore_axis_name="s", num_cores=NC, num_subcores=NS
)


@functools.partial(
    pl.kernel,
    out_type=jax.ShapeDtypeStruct((NC, NPAD, DW), jnp.float32),
    mesh=_MESH,
    scratch_types=[
        pltpu.VMEM((NCHUNK, CHUNK), jnp.int32),
        pltpu.VMEM((CHUNK, DW), jnp.float32),
        pltpu.VMEM((ROWS_PER_TILE, DW), jnp.float32),
        pltpu.VMEM_SHARED((NPAD, DW), jnp.float32),
    ],
    compiler_params=pltpu.CompilerParams(use_tc_tiling_on_sc=False),
)
def _deg_kernel(dst_hbm, out_hbm, dst_v, ones_v, stage_v, deg_sh):
    c = lax.axis_index("c")
    s = lax.axis_index("s")
    wid = c * NS + s
    base = s * ROWS_PER_TILE

    @pl.loop(0, CHUNK)
    def _(i):
        ones_v[i, :] = jnp.ones((DW,), jnp.float32)

    @pl.loop(0, ROWS_PER_TILE)
    def _(i):
        stage_v[i, :] = jnp.zeros((DW,), jnp.float32)

    pltpu.sync_copy(stage_v, deg_sh.at[pl.ds(base, ROWS_PER_TILE)])
    plsc.subcore_barrier()

    pltpu.sync_copy(dst_hbm.at[wid], dst_v)

    @pl.loop(0, NCHUNK)
    def _(g):
        pltpu.sync_copy(ones_v, deg_sh.at[dst_v.at[g]], add=True)

    plsc.subcore_barrier()
    pltpu.sync_copy(deg_sh.at[pl.ds(base, ROWS_PER_TILE)], stage_v)
    pltpu.sync_copy(stage_v, out_hbm.at[c, pl.ds(base, ROWS_PER_TILE)])


@functools.partial(
    pl.kernel,
    out_type=jax.ShapeDtypeStruct((NQ, NC, NPAD, HQ), jnp.float32),
    mesh=_MESH,
    scratch_types=[
        pltpu.VMEM((NCHUNK, CHUNK), jnp.int32),
        pltpu.VMEM((NCHUNK, CHUNK), jnp.int32),
        pltpu.VMEM((2, CHUNK, HQ), jnp.float32),
        pltpu.SemaphoreType.DMA,
        pltpu.SemaphoreType.DMA,
        pltpu.VMEM_SHARED((N_NODES, HQ), jnp.float32),
        pltpu.VMEM_SHARED((NPAD, HQ), jnp.float32),
    ],
    compiler_params=pltpu.CompilerParams(use_tc_tiling_on_sc=False,
                                         internal_scratch_in_bytes=0),
)
def _agg_kernel(hs0_hbm, hs1_hbm, src_hbm, dst_hbm, out_hbm,
                src_v, dst_v, buf_v, sem0, sem1, hs_sh, agg_sh):
    c = lax.axis_index("c")
    s = lax.axis_index("s")
    wid = c * NS + s
    base = s * ROWS_PER_TILE
    hs_base = s * HS_ROWS_PER_TILE
    sems = (sem0, sem1)
    rem = ROWS_PER_TILE - 4 * CHUNK

    pltpu.sync_copy(src_hbm.at[wid], src_v)
    pltpu.sync_copy(dst_hbm.at[wid], dst_v)

    for q, hs_hbm in enumerate((hs0_hbm, hs1_hbm)):
        @pl.loop(0, CHUNK)
        def _(i):
            for j in range(HQ // 16):
                buf_v[0, i, pl.ds(j * 16, 16)] = jnp.zeros((16,), jnp.float32)

        for k in range(4):
            pltpu.sync_copy(buf_v.at[0],
                            agg_sh.at[pl.ds(base + k * CHUNK, CHUNK)])
        pltpu.sync_copy(buf_v.at[0, pl.ds(0, rem)],
                        agg_sh.at[pl.ds(base + 4 * CHUNK, rem)])

        pltpu.sync_copy(hs_hbm.at[pl.ds(hs_base, HS_ROWS_PER_TILE)],
                        hs_sh.at[pl.ds(hs_base, HS_ROWS_PER_TILE)])
        plsc.subcore_barrier()

        pltpu.async_copy(hs_sh.at[src_v.at[0]], buf_v.at[0], sem0)
        pltpu.async_copy(hs_sh.at[src_v.at[1]], buf_v.at[1], sem1)

        @pl.loop(0, NCHUNK - 2, step=2)
        def _(g):
            for b in range(2):
                cur = g + b
                pltpu.make_async_copy(hs_sh.at[src_v.at[cur]], buf_v.at[b],
                                      sems[b]).wait()
                pltpu.sync_copy(buf_v.at[b], agg_sh.at[dst_v.at[cur]],
                                add=True)
                pltpu.async_copy(hs_sh.at[src_v.at[cur + 2]], buf_v.at[b],
                                 sems[b])

        for b in range(2):
            cur = NCHUNK - 2 + b
            pltpu.make_async_copy(hs_sh.at[src_v.at[cur]], buf_v.at[b],
                                  sems[b]).wait()
            pltpu.sync_copy(buf_v.at[b], agg_sh.at[dst_v.at[cur]], add=True)

        plsc.subcore_barrier()
        pltpu.sync_copy(agg_sh.at[pl.ds(base, ROWS_PER_TILE)],
                        out_hbm.at[q, c, pl.ds(base, ROWS_PER_TILE)])
        plsc.subcore_barrier()


def _hs_body(x_ref, w_ref, degp_ref, hs0_ref, hs1_ref):
    h = jnp.dot(x_ref[...], w_ref[...], preferred_element_type=jnp.float32)
    dp = degp_ref[...]
    deg = dp[0] + dp[1] + 1.0
    dinv = lax.rsqrt(deg)[:, 0:1]
    hs = h * dinv
    hs0_ref[...] = hs[:, 0 * HQ:1 * HQ]
    hs1_ref[...] = hs[:, 1 * HQ:2 * HQ]


def _out_body(aggp_ref, hs0_ref, hs1_ref, degp_ref, w2_ref,
              b1_ref, b2_ref, o_ref):
    dp = degp_ref[...]
    deg = dp[0] + dp[1] + 1.0
    dinv = lax.rsqrt(deg)[:, 0:1]
    ap = aggp_ref[...]
    agg = jnp.concatenate([ap[q, 0] + ap[q, 1] for q in range(NQ)], axis=1)
    hs = jnp.concatenate([hs0_ref[...], hs1_ref[...]], axis=1)
    t = (agg + hs) * dinv + b1_ref[...]
    o_ref[...] = jnp.dot(t, w2_ref[...], preferred_element_type=jnp.float32) \
        + b2_ref[...]


def kernel(edge_index, x, W1, b1, W2, b2):
    n, f = x.shape
    e = edge_index.shape[1]
    nclass = W2.shape[1]

    ei = edge_index.astype(jnp.int32)
    src = jnp.pad(ei[0], (0, E_PAD - e)).reshape(NW, NCHUNK, CHUNK)
    dst = jnp.pad(ei[1], (0, E_PAD - e),
                  constant_values=n).reshape(NW, NCHUNK, CHUNK)

    degp = _deg_kernel(dst)

    hsq = pl.pallas_call(
        _hs_body,
        grid=(n // RB,),
        in_specs=[
            pl.BlockSpec((RB, f), lambda i: (i, 0)),
            pl.BlockSpec((f, NHID), lambda i: (0, 0)),
            pl.BlockSpec((NC, RB, DW), lambda i: (0, i, 0)),
        ],
        out_specs=[pl.BlockSpec((RB, HQ), lambda i: (i, 0))] * NQ,
        out_shape=[jax.ShapeDtypeStruct((n, HQ), jnp.float32)] * NQ,
    )(x, W1, degp)

    aggp = _agg_kernel(*hsq, src, dst)

    w2p = jnp.zeros((NHID, 128), jnp.float32).at[:, :nclass].set(W2)
    b2p = jnp.zeros((128,), jnp.float32).at[:nclass].set(b2)

    out = pl.pallas_call(
        _out_body,
        grid=(n // RB,),
        in_specs=[
            pl.BlockSpec((NQ, NC, RB, HQ), lambda i: (0, 0, i, 0)),
            pl.BlockSpec((RB, HQ), lambda i: (i, 0)),
            pl.BlockSpec((RB, HQ), lambda i: (i, 0)),
            pl.BlockSpec((NC, RB, DW), lambda i: (0, i, 0)),
            pl.BlockSpec((NHID, 128), lambda i: (0, 0)),
            pl.BlockSpec((NHID,), lambda i: (0,)),
            pl.BlockSpec((128,), lambda i: (0,)),
        ],
        out_specs=pl.BlockSpec((RB, 128), lambda i: (i, 0)),
        out_shape=jax.ShapeDtypeStruct((n, 128), jnp.float32),
    )(aggp, *hsq, degp, w2p, b1, b2p)

    return out[:, :nclass]


# speedup vs baseline: 32.3038x; 1.1291x over previous
import functools

import jax
import jax.numpy as jnp
from jax import lax
from jax.experimental import pallas as pl
from jax.experimental.pallas import tpu as pltpu
from jax.experimental.pallas import tpu_sc as plsc

NC = 2
NS = 16
NW = NC * NS

NHID = 128
NQ = 2
HQ = NHID // NQ

CHUNK = 128
NCHUNK = 80
EDGES_PER_TILE = NCHUNK * CHUNK
E_PAD = NW * EDGES_PER_TILE

NPAD = 10112
ROWS_PER_TILE = NPAD // NS
N_NODES = 10000
HS_ROWS_PER_TILE = N_NODES // NS
DW = 16

RB = 2000

_MESH = plsc.VectorSubcoreMesh(
    core_axis_name="c", subcore_axis_name="s", num_cores=NC, num_subcores=NS
)
_SC_PARAMS = pltpu.CompilerParams(use_tc_tiling_on_sc=False,
                                  internal_scratch_in_bytes=0)


@functools.partial(
    pl.kernel,
    out_type=jax.ShapeDtypeStruct((NC, NPAD, DW), jnp.float32),
    mesh=_MESH,
    scratch_types=[
        pltpu.VMEM((NCHUNK, CHUNK), jnp.int32),
        pltpu.VMEM((CHUNK, DW), jnp.float32),
        pltpu.VMEM((ROWS_PER_TILE, DW), jnp.float32),
        pltpu.VMEM_SHARED((NPAD, DW), jnp.float32),
    ],
    compiler_params=_SC_PARAMS,
)
def _deg_kernel(dst_hbm, out_hbm, dst_v, ones_v, stage_v, deg_sh):
    c = lax.axis_index("c")
    s = lax.axis_index("s")
    wid = c * NS + s
    base = s * ROWS_PER_TILE

    @pl.loop(0, CHUNK)
    def _(i):
        ones_v[i, :] = jnp.ones((DW,), jnp.float32)

    @pl.loop(0, ROWS_PER_TILE)
    def _(i):
        stage_v[i, :] = jnp.zeros((DW,), jnp.float32)

    pltpu.sync_copy(stage_v, deg_sh.at[pl.ds(base, ROWS_PER_TILE)])
    plsc.subcore_barrier()

    pltpu.sync_copy(dst_hbm.at[wid], dst_v)

    @pl.loop(0, NCHUNK)
    def _(g):
        pltpu.sync_copy(ones_v, deg_sh.at[dst_v.at[g]], add=True)

    plsc.subcore_barrier()
    pltpu.sync_copy(deg_sh.at[pl.ds(base, ROWS_PER_TILE)], stage_v)
    pltpu.sync_copy(stage_v, out_hbm.at[c, pl.ds(base, ROWS_PER_TILE)])


@functools.partial(
    pl.kernel,
    out_type=jax.ShapeDtypeStruct((NC, NPAD, NHID), jnp.float32),
    mesh=_MESH,
    scratch_types=[
        pltpu.VMEM((NCHUNK, CHUNK), jnp.int32),
        pltpu.VMEM((NCHUNK, CHUNK), jnp.int32),
        pltpu.VMEM((2, CHUNK, HQ), jnp.float32),
        pltpu.SemaphoreType.DMA,
        pltpu.SemaphoreType.DMA,
        pltpu.VMEM_SHARED((N_NODES, HQ), jnp.float32),
        pltpu.VMEM_SHARED((NPAD, HQ), jnp.float32),
    ],
    compiler_params=_SC_PARAMS,
)
def _agg_kernel(hs_hbm, src_hbm, dst_hbm, out_hbm,
                src_v, dst_v, buf_v, sem0, sem1, hs_sh, agg_sh):
    c = lax.axis_index("c")
    s = lax.axis_index("s")
    wid = c * NS + s
    base = s * ROWS_PER_TILE
    hs_base = s * HS_ROWS_PER_TILE
    sems = (sem0, sem1)
    rem = ROWS_PER_TILE - 4 * CHUNK

    pltpu.sync_copy(src_hbm.at[wid], src_v)
    pltpu.sync_copy(dst_hbm.at[wid], dst_v)

    for q in range(NQ):
        @pl.loop(0, CHUNK)
        def _(i):
            for j in range(HQ // 16):
                buf_v[0, i, pl.ds(j * 16, 16)] = jnp.zeros((16,), jnp.float32)

        for k in range(4):
            pltpu.sync_copy(buf_v.at[0],
                            agg_sh.at[pl.ds(base + k * CHUNK, CHUNK)])
        pltpu.sync_copy(buf_v.at[0, pl.ds(0, rem)],
                        agg_sh.at[pl.ds(base + 4 * CHUNK, rem)])

        pltpu.sync_copy(
            hs_hbm.at[pl.ds(hs_base, HS_ROWS_PER_TILE), pl.ds(q * HQ, HQ)],
            hs_sh.at[pl.ds(hs_base, HS_ROWS_PER_TILE)])
        plsc.subcore_barrier()

        pltpu.async_copy(hs_sh.at[src_v.at[0]], buf_v.at[0], sem0)
        pltpu.async_copy(hs_sh.at[src_v.at[1]], buf_v.at[1], sem1)

        @pl.loop(0, NCHUNK - 2, step=2)
        def _(g):
            for b in range(2):
                cur = g + b
                pltpu.make_async_copy(hs_sh.at[src_v.at[cur]], buf_v.at[b],
                                      sems[b]).wait()
                pltpu.sync_copy(buf_v.at[b], agg_sh.at[dst_v.at[cur]],
                                add=True)
                pltpu.async_copy(hs_sh.at[src_v.at[cur + 2]], buf_v.at[b],
                                 sems[b])

        for b in range(2):
            cur = NCHUNK - 2 + b
            pltpu.make_async_copy(hs_sh.at[src_v.at[cur]], buf_v.at[b],
                                  sems[b]).wait()
            pltpu.sync_copy(buf_v.at[b], agg_sh.at[dst_v.at[cur]], add=True)

        plsc.subcore_barrier()
        pltpu.sync_copy(agg_sh.at[pl.ds(base, ROWS_PER_TILE)],
                        out_hbm.at[c, pl.ds(base, ROWS_PER_TILE),
                                   pl.ds(q * HQ, HQ)])
        plsc.subcore_barrier()


def _hs_body(x_ref, w_ref, degp_ref, hs_ref):
    h = jnp.dot(x_ref[...], w_ref[...], preferred_element_type=jnp.float32)
    dp = degp_ref[...]
    deg = dp[0] + dp[1] + 1.0
    dinv = lax.rsqrt(deg)[:, 0:1]
    hs_ref[...] = h * dinv


def _out_body(aggp_ref, hs_ref, degp_ref, w2_ref, b1_ref, b2_ref, o_ref):
    dp = degp_ref[...]
    deg = dp[0] + dp[1] + 1.0
    dinv = lax.rsqrt(deg)[:, 0:1]
    ap = aggp_ref[...]
    t = (ap[0] + ap[1] + hs_ref[...]) * dinv + b1_ref[...]
    res = jnp.dot(t, w2_ref[...], preferred_element_type=jnp.float32) \
        + b2_ref[...]
    o_ref[...] = res[:, :o_ref.shape[1]]


def kernel(edge_index, x, W1, b1, W2, b2):
    n, f = x.shape
    e = edge_index.shape[1]
    nclass = W2.shape[1]

    ei = edge_index.astype(jnp.int32)
    src = jnp.pad(ei[0], (0, E_PAD - e)).reshape(NW, NCHUNK, CHUNK)
    dst = jnp.pad(ei[1], (0, E_PAD - e),
                  constant_values=n).reshape(NW, NCHUNK, CHUNK)

    degp = _deg_kernel(dst)

    hs = pl.pallas_call(
        _hs_body,
        grid=(n // RB,),
        in_specs=[
            pl.BlockSpec((RB, f), lambda i: (i, 0)),
            pl.BlockSpec((f, NHID), lambda i: (0, 0)),
            pl.BlockSpec((NC, RB, DW), lambda i: (0, i, 0)),
        ],
        out_specs=pl.BlockSpec((RB, NHID), lambda i: (i, 0)),
        out_shape=jax.ShapeDtypeStruct((n, NHID), jnp.float32),
    )(x, W1, degp)

    aggp = _agg_kernel(hs, src, dst)

    w2p = jnp.zeros((NHID, 128), jnp.float32).at[:, :nclass].set(W2)
    b2p = jnp.zeros((128,), jnp.float32).at[:nclass].set(b2)

    out = pl.pallas_call(
        _out_body,
        grid=(n // RB,),
        in_specs=[
            pl.BlockSpec((NC, RB, NHID), lambda i: (0, i, 0)),
            pl.BlockSpec((RB, NHID), lambda i: (i, 0)),
            pl.BlockSpec((NC, RB, DW), lambda i: (0, i, 0)),
            pl.BlockSpec((NHID, 128), lambda i: (0, 0)),
            pl.BlockSpec((NHID,), lambda i: (0,)),
            pl.BlockSpec((128,), lambda i: (0,)),
        ],
        out_specs=pl.BlockSpec((RB, nclass), lambda i: (i, 0)),
        out_shape=jax.ShapeDtypeStruct((n, nclass), jnp.float32),
    )(aggp, hs, degp, w2p, b1, b2p)

    return out


# speedup vs baseline: 33.0367x; 1.0227x over previous
import functools

import jax
import jax.numpy as jnp
from jax import lax
from jax.experimental import pallas as pl
from jax.experimental.pallas import tpu as pltpu
from jax.experimental.pallas import tpu_sc as plsc

NC = 2
NS = 16
NW = NC * NS

NHID = 128
NQ = 2
HQ = NHID // NQ

CHUNK = 125
NCHUNK = 80

NPAD = 10112
ROWS_PER_TILE = NPAD // NS
N_NODES = 10000
HS_ROWS_PER_TILE = N_NODES // NS
DW = 16

RB = 2000

_MESH = plsc.VectorSubcoreMesh(
    core_axis_name="c", subcore_axis_name="s", num_cores=NC, num_subcores=NS
)
_SC_PARAMS = pltpu.CompilerParams(use_tc_tiling_on_sc=False,
                                  internal_scratch_in_bytes=0)


@functools.partial(
    pl.kernel,
    out_type=jax.ShapeDtypeStruct((NC, NPAD, DW), jnp.float32),
    mesh=_MESH,
    scratch_types=[
        pltpu.VMEM((NCHUNK, CHUNK), jnp.int32),
        pltpu.VMEM((CHUNK, DW), jnp.float32),
        pltpu.VMEM((ROWS_PER_TILE, DW), jnp.float32),
        pltpu.SemaphoreType.DMA,
        pltpu.SemaphoreType.DMA,
        pltpu.VMEM_SHARED((NPAD, DW), jnp.float32),
    ],
    compiler_params=_SC_PARAMS,
)
def _deg_kernel(dst_hbm, out_hbm, dst_v, ones_v, stage_v, sem0, sem1, deg_sh):
    c = lax.axis_index("c")
    s = lax.axis_index("s")
    wid = c * NS + s
    base = s * ROWS_PER_TILE
    sems = (sem0, sem1)

    @pl.loop(0, CHUNK)
    def _(i):
        ones_v[i, :] = jnp.ones((DW,), jnp.float32)

    @pl.loop(0, ROWS_PER_TILE)
    def _(i):
        stage_v[i, :] = jnp.zeros((DW,), jnp.float32)

    pltpu.sync_copy(stage_v, deg_sh.at[pl.ds(base, ROWS_PER_TILE)])
    plsc.subcore_barrier()

    pltpu.sync_copy(dst_hbm.at[wid], dst_v)

    pltpu.async_copy(ones_v, deg_sh.at[dst_v.at[0]], sem0, add=True)
    pltpu.async_copy(ones_v, deg_sh.at[dst_v.at[1]], sem1, add=True)

    @pl.loop(0, NCHUNK - 2, step=2)
    def _(g):
        for b in range(2):
            cur = g + b
            pltpu.make_async_copy(ones_v, deg_sh.at[dst_v.at[cur]],
                                  sems[b]).wait()
            pltpu.async_copy(ones_v, deg_sh.at[dst_v.at[cur + 2]], sems[b],
                             add=True)

    for b in range(2):
        cur = NCHUNK - 2 + b
        pltpu.make_async_copy(ones_v, deg_sh.at[dst_v.at[cur]],
                              sems[b]).wait()

    plsc.subcore_barrier()
    pltpu.sync_copy(deg_sh.at[pl.ds(base, ROWS_PER_TILE)], stage_v)
    pltpu.sync_copy(stage_v, out_hbm.at[c, pl.ds(base, ROWS_PER_TILE)])


@functools.partial(
    pl.kernel,
    out_type=jax.ShapeDtypeStruct((NC, NPAD, NHID), jnp.float32),
    mesh=_MESH,
    scratch_types=[
        pltpu.VMEM((NCHUNK, CHUNK), jnp.int32),
        pltpu.VMEM((NCHUNK, CHUNK), jnp.int32),
        pltpu.VMEM((2, CHUNK, HQ), jnp.float32),
        pltpu.SemaphoreType.DMA,
        pltpu.SemaphoreType.DMA,
        pltpu.VMEM_SHARED((N_NODES, HQ), jnp.float32),
        pltpu.VMEM_SHARED((NPAD, HQ), jnp.float32),
    ],
    compiler_params=_SC_PARAMS,
)
def _agg_kernel(hs_hbm, src_hbm, dst_hbm, out_hbm,
                src_v, dst_v, buf_v, sem0, sem1, hs_sh, agg_sh):
    c = lax.axis_index("c")
    s = lax.axis_index("s")
    wid = c * NS + s
    base = s * ROWS_PER_TILE
    hs_base = s * HS_ROWS_PER_TILE
    sems = (sem0, sem1)
    nz = ROWS_PER_TILE // CHUNK
    rem = ROWS_PER_TILE - nz * CHUNK

    pltpu.sync_copy(src_hbm.at[wid], src_v)
    pltpu.sync_copy(dst_hbm.at[wid], dst_v)

    for q in range(NQ):
        @pl.loop(0, CHUNK)
        def _(i):
            for j in range(HQ // 16):
                buf_v[0, i, pl.ds(j * 16, 16)] = jnp.zeros((16,), jnp.float32)

        for k in range(nz):
            pltpu.sync_copy(buf_v.at[0],
                            agg_sh.at[pl.ds(base + k * CHUNK, CHUNK)])
        pltpu.sync_copy(buf_v.at[0, pl.ds(0, rem)],
                        agg_sh.at[pl.ds(base + nz * CHUNK, rem)])

        pltpu.sync_copy(
            hs_hbm.at[pl.ds(hs_base, HS_ROWS_PER_TILE), pl.ds(q * HQ, HQ)],
            hs_sh.at[pl.ds(hs_base, HS_ROWS_PER_TILE)])
        plsc.subcore_barrier()

        pltpu.async_copy(hs_sh.at[src_v.at[0]], buf_v.at[0], sem0)
        pltpu.async_copy(hs_sh.at[src_v.at[1]], buf_v.at[1], sem1)

        @pl.loop(0, NCHUNK - 2, step=2)
        def _(g):
            for b in range(2):
                cur = g + b
                pltpu.make_async_copy(hs_sh.at[src_v.at[cur]], buf_v.at[b],
                                      sems[b]).wait()
                pltpu.sync_copy(buf_v.at[b], agg_sh.at[dst_v.at[cur]],
                                add=True)
                pltpu.async_copy(hs_sh.at[src_v.at[cur + 2]], buf_v.at[b],
                                 sems[b])

        for b in range(2):
            cur = NCHUNK - 2 + b
            pltpu.make_async_copy(hs_sh.at[src_v.at[cur]], buf_v.at[b],
                                  sems[b]).wait()
            pltpu.sync_copy(buf_v.at[b], agg_sh.at[dst_v.at[cur]], add=True)

        plsc.subcore_barrier()
        pltpu.sync_copy(agg_sh.at[pl.ds(base, ROWS_PER_TILE)],
                        out_hbm.at[c, pl.ds(base, ROWS_PER_TILE),
                                   pl.ds(q * HQ, HQ)])
        plsc.subcore_barrier()


def _mm_body(x_ref, w_ref, h_ref):
    h_ref[...] = jnp.dot(x_ref[...], w_ref[...],
                         preferred_element_type=jnp.float32)


def _scale_body(h_ref, degp_ref, hs_ref):
    dp = degp_ref[...]
    deg = dp[0] + dp[1] + 1.0
    dinv = lax.rsqrt(deg)[:, 0:1]
    hs_ref[...] = h_ref[...] * dinv


def _out_body(aggp_ref, hs_ref, degp_ref, w2_ref, b1_ref, b2_ref, o_ref):
    dp = degp_ref[...]
    deg = dp[0] + dp[1] + 1.0
    dinv = lax.rsqrt(deg)[:, 0:1]
    ap = aggp_ref[...]
    t = (ap[0] + ap[1] + hs_ref[...]) * dinv + b1_ref[...]
    res = jnp.dot(t, w2_ref[...], preferred_element_type=jnp.float32) \
        + b2_ref[...]
    o_ref[...] = res[:, :o_ref.shape[1]]


def kernel(edge_index, x, W1, b1, W2, b2):
    n, f = x.shape
    nclass = W2.shape[1]

    ei = edge_index.astype(jnp.int32).reshape(2, NW, NCHUNK, CHUNK)
    src = ei[0]
    dst = ei[1]

    degp = _deg_kernel(dst)

    h = pl.pallas_call(
        _mm_body,
        grid=(n // RB,),
        in_specs=[
            pl.BlockSpec((RB, f), lambda i: (i, 0)),
            pl.BlockSpec((f, NHID), lambda i: (0, 0)),
        ],
        out_specs=pl.BlockSpec((RB, NHID), lambda i: (i, 0)),
        out_shape=jax.ShapeDtypeStruct((n, NHID), jnp.float32),
    )(x, W1)

    hs = pl.pallas_call(
        _scale_body,
        grid=(n // RB,),
        in_specs=[
            pl.BlockSpec((RB, NHID), lambda i: (i, 0)),
            pl.BlockSpec((NC, RB, DW), lambda i: (0, i, 0)),
        ],
        out_specs=pl.BlockSpec((RB, NHID), lambda i: (i, 0)),
        out_shape=jax.ShapeDtypeStruct((n, NHID), jnp.float32),
    )(h, degp)

    aggp = _agg_kernel(hs, src, dst)

    w2p = jnp.zeros((NHID, 128), jnp.float32).at[:, :nclass].set(W2)
    b2p = jnp.zeros((128,), jnp.float32).at[:nclass].set(b2)

    out = pl.pallas_call(
        _out_body,
        grid=(n // RB,),
        in_specs=[
            pl.BlockSpec((NC, RB, NHID), lambda i: (0, i, 0)),
            pl.BlockSpec((RB, NHID), lambda i: (i, 0)),
            pl.BlockSpec((NC, RB, DW), lambda i: (0, i, 0)),
            pl.BlockSpec((NHID, 128), lambda i: (0, 0)),
            pl.BlockSpec((NHID,), lambda i: (0,)),
            pl.BlockSpec((128,), lambda i: (0,)),
        ],
        out_specs=pl.BlockSpec((RB, nclass), lambda i: (i, 0)),
        out_shape=jax.ShapeDtypeStruct((n, nclass), jnp.float32),
    )(aggp, hs, degp, w2p, b1, b2p)

    return out


# speedup vs baseline: 33.9034x; 1.0262x over previous
import functools

import jax
import jax.numpy as jnp
from jax import lax
from jax.experimental import pallas as pl
from jax.experimental.pallas import tpu as pltpu
from jax.experimental.pallas import tpu_sc as plsc

NC = 2
NS = 16
NW = NC * NS

NHID = 128
NQ = 2
HQ = NHID // NQ

CHUNK = 125
NCHUNK = 80

NPAD = 10000
ROWS_PER_TILE = NPAD // NS
N_NODES = 10000
HS_ROWS_PER_TILE = N_NODES // NS
DW = 8

RB = 2000

_MESH = plsc.VectorSubcoreMesh(
    core_axis_name="c", subcore_axis_name="s", num_cores=NC, num_subcores=NS
)
_SC_PARAMS = pltpu.CompilerParams(use_tc_tiling_on_sc=False,
                                  internal_scratch_in_bytes=0)


@functools.partial(
    pl.kernel,
    out_type=jax.ShapeDtypeStruct((NC, NPAD, DW), jnp.float32),
    mesh=_MESH,
    scratch_types=[
        pltpu.VMEM((NCHUNK, CHUNK), jnp.int32),
        pltpu.VMEM((CHUNK, DW), jnp.float32),
        pltpu.VMEM((ROWS_PER_TILE, DW), jnp.float32),
        pltpu.SemaphoreType.DMA,
        pltpu.SemaphoreType.DMA,
        pltpu.VMEM_SHARED((NPAD, DW), jnp.float32),
    ],
    compiler_params=_SC_PARAMS,
)
def _deg_kernel(dst_hbm, out_hbm, dst_v, ones_v, stage_v, sem0, sem1, deg_sh):
    c = lax.axis_index("c")
    s = lax.axis_index("s")
    wid = c * NS + s
    base = s * ROWS_PER_TILE
    sems = (sem0, sem1)

    @pl.loop(0, CHUNK)
    def _(i):
        ones_v[i, :] = jnp.ones((DW,), jnp.float32)

    @pl.loop(0, ROWS_PER_TILE)
    def _(i):
        stage_v[i, :] = jnp.zeros((DW,), jnp.float32)

    pltpu.sync_copy(stage_v, deg_sh.at[pl.ds(base, ROWS_PER_TILE)])
    plsc.subcore_barrier()

    pltpu.sync_copy(dst_hbm.at[wid], dst_v)

    pltpu.async_copy(ones_v, deg_sh.at[dst_v.at[0]], sem0, add=True)
    pltpu.async_copy(ones_v, deg_sh.at[dst_v.at[1]], sem1, add=True)

    @pl.loop(0, NCHUNK - 2, step=2)
    def _(g):
        for b in range(2):
            cur = g + b
            pltpu.make_async_copy(ones_v, deg_sh.at[dst_v.at[cur]],
                                  sems[b]).wait()
            pltpu.async_copy(ones_v, deg_sh.at[dst_v.at[cur + 2]], sems[b],
                             add=True)

    for b in range(2):
        cur = NCHUNK - 2 + b
        pltpu.make_async_copy(ones_v, deg_sh.at[dst_v.at[cur]],
                              sems[b]).wait()

    plsc.subcore_barrier()
    pltpu.sync_copy(deg_sh.at[pl.ds(base, ROWS_PER_TILE)], stage_v)
    pltpu.sync_copy(stage_v, out_hbm.at[c, pl.ds(base, ROWS_PER_TILE)])


@functools.partial(
    pl.kernel,
    out_type=jax.ShapeDtypeStruct((NC, NPAD, NHID), jnp.float32),
    mesh=_MESH,
    scratch_types=[
        pltpu.VMEM((NCHUNK, CHUNK), jnp.int32),
        pltpu.VMEM((NCHUNK, CHUNK), jnp.int32),
        pltpu.VMEM((2, CHUNK, HQ), jnp.float32),
        pltpu.SemaphoreType.DMA,
        pltpu.SemaphoreType.DMA,
        pltpu.SemaphoreType.DMA,
        pltpu.SemaphoreType.DMA,
        pltpu.VMEM_SHARED((N_NODES, HQ), jnp.float32),
        pltpu.VMEM_SHARED((NPAD, HQ), jnp.float32),
    ],
    compiler_params=_SC_PARAMS,
)
def _agg_kernel(hs_hbm, src_hbm, dst_hbm, out_hbm,
                src_v, dst_v, buf_v, g0, g1, s0, s1, hs_sh, agg_sh):
    c = lax.axis_index("c")
    s = lax.axis_index("s")
    wid = c * NS + s
    base = s * ROWS_PER_TILE
    hs_base = s * HS_ROWS_PER_TILE
    gsems = (g0, g1)
    ssems = (s0, s1)
    nz = ROWS_PER_TILE // CHUNK

    pltpu.sync_copy(src_hbm.at[wid], src_v)
    pltpu.sync_copy(dst_hbm.at[wid], dst_v)

    for q in range(NQ):
        @pl.loop(0, CHUNK)
        def _(i):
            for j in range(HQ // 16):
                buf_v[0, i, pl.ds(j * 16, 16)] = jnp.zeros((16,), jnp.float32)

        for k in range(nz):
            pltpu.sync_copy(buf_v.at[0],
                            agg_sh.at[pl.ds(base + k * CHUNK, CHUNK)])

        pltpu.sync_copy(
            hs_hbm.at[pl.ds(hs_base, HS_ROWS_PER_TILE), pl.ds(q * HQ, HQ)],
            hs_sh.at[pl.ds(hs_base, HS_ROWS_PER_TILE)])
        plsc.subcore_barrier()

        for b in range(2):
            pltpu.async_copy(hs_sh.at[src_v.at[b]], buf_v.at[b], gsems[b])

        @pl.loop(0, NCHUNK - 2, step=2)
        def _(g):
            for b in range(2):
                cur = g + b
                pltpu.make_async_copy(hs_sh.at[src_v.at[cur]], buf_v.at[b],
                                      gsems[b]).wait()
                pltpu.async_copy(buf_v.at[b], agg_sh.at[dst_v.at[cur]],
                                 ssems[b], add=True)
                pltpu.make_async_copy(buf_v.at[b], agg_sh.at[dst_v.at[cur]],
                                      ssems[b]).wait()
                pltpu.async_copy(hs_sh.at[src_v.at[cur + 2]], buf_v.at[b],
                                 gsems[b])

        for b in range(2):
            cur = NCHUNK - 2 + b
            pltpu.make_async_copy(hs_sh.at[src_v.at[cur]], buf_v.at[b],
                                  gsems[b]).wait()
            pltpu.async_copy(buf_v.at[b], agg_sh.at[dst_v.at[cur]],
                             ssems[b], add=True)
        for b in range(2):
            cur = NCHUNK - 2 + b
            pltpu.make_async_copy(buf_v.at[b], agg_sh.at[dst_v.at[cur]],
                                  ssems[b]).wait()

        plsc.subcore_barrier()
        pltpu.sync_copy(agg_sh.at[pl.ds(base, ROWS_PER_TILE)],
                        out_hbm.at[c, pl.ds(base, ROWS_PER_TILE),
                                   pl.ds(q * HQ, HQ)])
        plsc.subcore_barrier()


def _hs_body(x_ref, w_ref, degp_ref, hs_ref):
    h = jnp.dot(x_ref[...], w_ref[...], preferred_element_type=jnp.float32)
    dp = degp_ref[...]
    deg = dp[0] + dp[1] + 1.0
    dinv = lax.rsqrt(deg)[:, 0:1]
    hs_ref[...] = h * dinv


def _out_body(aggp_ref, hs_ref, degp_ref, w2_ref, b1_ref, b2_ref, o_ref):
    dp = degp_ref[...]
    deg = dp[0] + dp[1] + 1.0
    dinv = lax.rsqrt(deg)[:, 0:1]
    ap = aggp_ref[...]
    t = (ap[0] + ap[1] + hs_ref[...]) * dinv + b1_ref[...]
    res = jnp.dot(t, w2_ref[...], preferred_element_type=jnp.float32) \
        + b2_ref[...]
    o_ref[...] = res[:, :o_ref.shape[1]]


def kernel(edge_index, x, W1, b1, W2, b2):
    n, f = x.shape
    nclass = W2.shape[1]

    ei = edge_index.astype(jnp.int32).reshape(2, NW, NCHUNK, CHUNK)
    src = ei[0]
    dst = ei[1]

    degp = _deg_kernel(dst)

    hs = pl.pallas_call(
        _hs_body,
        grid=(n // RB,),
        in_specs=[
            pl.BlockSpec((RB, f), lambda i: (i, 0)),
            pl.BlockSpec((f, NHID), lambda i: (0, 0)),
            pl.BlockSpec((NC, RB, DW), lambda i: (0, i, 0)),
        ],
        out_specs=pl.BlockSpec((RB, NHID), lambda i: (i, 0)),
        out_shape=jax.ShapeDtypeStruct((n, NHID), jnp.float32),
    )(x, W1, degp)

    aggp = _agg_kernel(hs, src, dst)

    w2p = jnp.zeros((NHID, 128), jnp.float32).at[:, :nclass].set(W2)
    b2p = jnp.zeros((128,), jnp.float32).at[:nclass].set(b2)

    out = pl.pallas_call(
        _out_body,
        grid=(n // RB,),
        in_specs=[
            pl.BlockSpec((NC, RB, NHID), lambda i: (0, i, 0)),
            pl.BlockSpec((RB, NHID), lambda i: (i, 0)),
            pl.BlockSpec((NC, RB, DW), lambda i: (0, i, 0)),
            pl.BlockSpec((NHID, 128), lambda i: (0, 0)),
            pl.BlockSpec((NHID,), lambda i: (0,)),
            pl.BlockSpec((128,), lambda i: (0,)),
        ],
        out_specs=pl.BlockSpec((RB, nclass), lambda i: (i, 0)),
        out_shape=jax.ShapeDtypeStruct((n, nclass), jnp.float32),
    )(aggp, hs, degp, w2p, b1, b2p)

    return out
